# Initial kernel scaffold; baseline (speedup 1.0000x reference)
import jax
import jax.numpy as jnp
from jax.experimental import pallas as pl


def kernel(txt_x, W_enc):
    raise NotImplementedError("write your pallas kernel here")


# speedup vs baseline: 20.4663x; 20.4663x over previous
import jax
import jax.numpy as jnp
from jax.experimental import pallas as pl
from jax.experimental.pallas import tpu as pltpu

K_SEL = 256
LN_EPS = 1e-5
_PREC = jax.lax.Precision.DEFAULT


def _encode_body(x_ref, w_ref, lat_ref, key_ref, mu_ref, std_ref, xn_ref):
    step = pl.program_id(0)
    d = x_ref.shape[1]

    @pl.when(step == 0)
    def _():
        x = x_ref[...]
        mu = jnp.mean(x, axis=1, keepdims=True)
        xc = x - mu
        m2 = jnp.mean(xc, axis=1, keepdims=True)
        var = jnp.sum((xc - m2) * (xc - m2), axis=1, keepdims=True) / (d - 1)
        std = jnp.sqrt(var)
        xn_ref[...] = xc / (std + LN_EPS)
        mu_ref[...] = mu
        std_ref[...] = std

    w = w_ref[...]
    w2 = jnp.sum(w * w, axis=1, keepdims=True)
    inv = 1.0 / jnp.maximum(jnp.sqrt(w2), 1e-12)
    wn = w * inv
    lat = jax.lax.dot_general(
        xn_ref[...], wn, (((1,), (1,)), ((), ())),
        precision=_PREC, preferred_element_type=jnp.float32)
    lat_ref[...] = lat
    i = jax.lax.bitcast_convert_type(lat, jnp.int32)
    key_ref[...] = i ^ (jax.lax.shift_right_arithmetic(i, 31) & 0x7FFFFFFF)


def _select_body(key_ref, thr_ref):
    b, h = key_ref.shape
    n_chunks = 16
    ch = h // n_chunks

    def count_gt(mid):
        c = jnp.zeros((b, 1), jnp.int32)
        for j in range(n_chunks):
            k = key_ref[:, pl.ds(j * ch, ch)]
            c = c + jnp.sum((k > mid).astype(jnp.int32), axis=1,
                            keepdims=True)
        return c

    lo0 = jnp.full((b, 1), jnp.iinfo(jnp.int32).min, jnp.int32)
    hi0 = jnp.full((b, 1), jnp.iinfo(jnp.int32).max, jnp.int32)

    def body(_, carry):
        lo, hi = carry
        mid = (lo & hi) + jax.lax.shift_right_arithmetic(lo ^ hi, 1)
        c = count_gt(mid)
        live = lo < hi
        hi = jnp.where(live & (c < K_SEL), mid, hi)
        lo = jnp.where(live & (c >= K_SEL), mid + 1, lo)
        return lo, hi

    lo, hi = jax.lax.fori_loop(0, 32, body, (lo0, hi0))
    u0 = hi
    i0 = jnp.where(u0 >= 0, u0, u0 ^ 0x7FFFFFFF)
    thr_ref[...] = jax.lax.bitcast_convert_type(i0, jnp.float32)


def _decode_body(lat_ref, w_ref, thr_ref, mu_ref, std_ref, out_ref, acc_ref):
    step = pl.program_id(0)
    nt = pl.num_programs(0)

    @pl.when(step == 0)
    def _():
        acc_ref[...] = jnp.zeros_like(acc_ref)

    lat = lat_ref[...]
    masked = jnp.where(lat >= thr_ref[...], lat, 0.0)
    acc_ref[...] += jax.lax.dot_general(
        masked, w_ref[...], (((1,), (0,)), ((), ())),
        precision=_PREC, preferred_element_type=jnp.float32)

    @pl.when(step == nt - 1)
    def _():
        out_ref[...] = acc_ref[...] * std_ref[...] + mu_ref[...]


def kernel(txt_x, W_enc):
    b, d = txt_x.shape
    h = W_enc.shape[0]
    th = 2048 if h % 2048 == 0 else h
    nt = h // th
    f32 = jnp.float32

    lat, keys, mu, std = pl.pallas_call(
        _encode_body,
        grid=(nt,),
        in_specs=[
            pl.BlockSpec((b, d), lambda i: (0, 0)),
            pl.BlockSpec((th, d), lambda i: (i, 0)),
        ],
        out_specs=[
            pl.BlockSpec((b, th), lambda i: (0, i)),
            pl.BlockSpec((b, th), lambda i: (0, i)),
            pl.BlockSpec((b, 1), lambda i: (0, 0)),
            pl.BlockSpec((b, 1), lambda i: (0, 0)),
        ],
        out_shape=[
            jax.ShapeDtypeStruct((b, h), f32),
            jax.ShapeDtypeStruct((b, h), jnp.int32),
            jax.ShapeDtypeStruct((b, 1), f32),
            jax.ShapeDtypeStruct((b, 1), f32),
        ],
        scratch_shapes=[pltpu.VMEM((b, d), f32)],
    )(txt_x, W_enc)

    thr = pl.pallas_call(
        _select_body,
        out_shape=jax.ShapeDtypeStruct((b, 1), f32),
    )(keys)

    recon = pl.pallas_call(
        _decode_body,
        grid=(nt,),
        in_specs=[
            pl.BlockSpec((b, th), lambda i: (0, i)),
            pl.BlockSpec((th, d), lambda i: (i, 0)),
            pl.BlockSpec((b, 1), lambda i: (0, 0)),
            pl.BlockSpec((b, 1), lambda i: (0, 0)),
            pl.BlockSpec((b, 1), lambda i: (0, 0)),
        ],
        out_specs=pl.BlockSpec((b, d), lambda i: (0, 0)),
        out_shape=jax.ShapeDtypeStruct((b, d), f32),
        scratch_shapes=[pltpu.VMEM((b, d), f32)],
    )(lat, W_enc, thr, mu, std)

    return (recon, lat)


# speedup vs baseline: 25.3173x; 1.2370x over previous
import jax
import jax.numpy as jnp
from jax.experimental import pallas as pl
from jax.experimental.pallas import tpu as pltpu

K_SEL = 256
LN_EPS = 1e-5
_PREC = jax.lax.Precision.DEFAULT
_N_ITERS = 18


def _encode_body(x_ref, w_ref, lat_ref, mn_ref, mx_ref, mu_ref, std_ref,
                 xn_ref):
    step = pl.program_id(0)
    d = x_ref.shape[1]

    @pl.when(step == 0)
    def _():
        x = x_ref[...]
        mu = jnp.mean(x, axis=1, keepdims=True)
        xc = x - mu
        m2 = jnp.mean(xc, axis=1, keepdims=True)
        var = jnp.sum((xc - m2) * (xc - m2), axis=1, keepdims=True) / (d - 1)
        std = jnp.sqrt(var)
        xn_ref[...] = xc / (std + LN_EPS)
        mu_ref[...] = mu
        std_ref[...] = std

    w = w_ref[...]
    w2 = jnp.sum(w * w, axis=1, keepdims=True)
    inv = 1.0 / jnp.maximum(jnp.sqrt(w2), 1e-12)
    wn = w * inv
    lat = jax.lax.dot_general(
        xn_ref[...], wn, (((1,), (1,)), ((), ())),
        precision=_PREC, preferred_element_type=jnp.float32)
    lat_ref[...] = lat
    tmin = jnp.min(lat, axis=1, keepdims=True)
    tmax = jnp.max(lat, axis=1, keepdims=True)

    @pl.when(step == 0)
    def _():
        mn_ref[...] = tmin
        mx_ref[...] = tmax

    @pl.when(step > 0)
    def _():
        mn_ref[...] = jnp.minimum(mn_ref[...], tmin)
        mx_ref[...] = jnp.maximum(mx_ref[...], tmax)


def _select_body(lat_ref, mn_ref, mx_ref, thr_ref):
    b, h = lat_ref.shape
    n_chunks = 16
    ch = h // n_chunks

    def count_ge(t):
        c = jnp.zeros((b, 1), jnp.int32)
        for j in range(n_chunks):
            v = lat_ref[:, pl.ds(j * ch, ch)]
            c = c + jnp.sum((v >= t).astype(jnp.int32), axis=1,
                            keepdims=True)
        return c

    mn = mn_ref[...]
    mx = mx_ref[...]
    lo = mn
    hi = mx + (jnp.abs(mx) * 1e-6 + 1e-30)
    c_lo = jnp.full((b, 1), h, jnp.int32)
    c_hi = jnp.zeros((b, 1), jnp.int32)

    for _ in range(_N_ITERS):
        c_lo_f = c_lo.astype(jnp.float32)
        c_hi_f = jnp.maximum(c_hi.astype(jnp.float32), 0.7)
        frac = jnp.log(c_lo_f / K_SEL) / jnp.log(c_lo_f / c_hi_f)
        frac = jnp.clip(frac, 0.015625, 0.984375)
        m0 = lo + (hi - lo) * frac
        m = jnp.where((m0 > lo) & (m0 < hi), m0, 0.5 * (lo + hi))
        c = count_ge(m)
        go_hi = c <= K_SEL
        go_lo = c > K_SEL
        hi = jnp.where(go_hi, m, hi)
        c_hi = jnp.where(go_hi, c, c_hi)
        lo = jnp.where(go_lo, m, lo)
        c_lo = jnp.where(go_lo, c, c_lo)

    thr_ref[...] = hi


def _decode_body(lat_ref, w_ref, thr_ref, mu_ref, std_ref, out_ref, acc_ref):
    step = pl.program_id(0)
    nt = pl.num_programs(0)

    @pl.when(step == 0)
    def _():
        acc_ref[...] = jnp.zeros_like(acc_ref)

    lat = lat_ref[...]
    masked = jnp.where(lat >= thr_ref[...], lat, 0.0)
    acc_ref[...] += jax.lax.dot_general(
        masked, w_ref[...], (((1,), (0,)), ((), ())),
        precision=_PREC, preferred_element_type=jnp.float32)

    @pl.when(step == nt - 1)
    def _():
        out_ref[...] = acc_ref[...] * std_ref[...] + mu_ref[...]


def kernel(txt_x, W_enc):
    b, d = txt_x.shape
    h = W_enc.shape[0]
    th = 2048 if h % 2048 == 0 else h
    nt = h // th
    f32 = jnp.float32

    lat, mn, mx, mu, std = pl.pallas_call(
        _encode_body,
        grid=(nt,),
        in_specs=[
            pl.BlockSpec((b, d), lambda i: (0, 0)),
            pl.BlockSpec((th, d), lambda i: (i, 0)),
        ],
        out_specs=[
            pl.BlockSpec((b, th), lambda i: (0, i)),
            pl.BlockSpec((b, 1), lambda i: (0, 0)),
            pl.BlockSpec((b, 1), lambda i: (0, 0)),
            pl.BlockSpec((b, 1), lambda i: (0, 0)),
            pl.BlockSpec((b, 1), lambda i: (0, 0)),
        ],
        out_shape=[
            jax.ShapeDtypeStruct((b, h), f32),
            jax.ShapeDtypeStruct((b, 1), f32),
            jax.ShapeDtypeStruct((b, 1), f32),
            jax.ShapeDtypeStruct((b, 1), f32),
            jax.ShapeDtypeStruct((b, 1), f32),
        ],
        scratch_shapes=[pltpu.VMEM((b, d), f32)],
    )(txt_x, W_enc)

    thr = pl.pallas_call(
        _select_body,
        out_shape=jax.ShapeDtypeStruct((b, 1), f32),
    )(lat, mn, mx)

    recon = pl.pallas_call(
        _decode_body,
        grid=(nt,),
        in_specs=[
            pl.BlockSpec((b, th), lambda i: (0, i)),
            pl.BlockSpec((th, d), lambda i: (i, 0)),
            pl.BlockSpec((b, 1), lambda i: (0, 0)),
            pl.BlockSpec((b, 1), lambda i: (0, 0)),
            pl.BlockSpec((b, 1), lambda i: (0, 0)),
        ],
        out_specs=pl.BlockSpec((b, d), lambda i: (0, 0)),
        out_shape=jax.ShapeDtypeStruct((b, d), f32),
        scratch_shapes=[pltpu.VMEM((b, d), f32)],
    )(lat, W_enc, thr, mu, std)

    return (recon, lat)
